# baseline (speedup 1.0000x reference)
import functools

import jax
import jax.numpy as jnp
from jax import lax
from jax.experimental import pallas as pl
from jax.experimental.pallas import tpu as pltpu
from jax.experimental.pallas import tpu_sc as plsc

N = 10000
E = 320000
F_IN = 128
D_EDGE = 4
KN = 32
F1 = 64
F2 = 64
W80 = 80

NC = 2
NS = 16
NW = NC * NS
EPW = E // NW
CH = 400
NCH = EPW // CH
RPT = N // NS

_mesh = plsc.VectorSubcoreMesh(core_axis_name="c", subcore_axis_name="s")
EB = 512
GB = E // EB


def _wid():
    return lax.axis_index("s") * NC + lax.axis_index("c")


@functools.partial(
    pl.kernel, mesh=_mesh,
    compiler_params=pltpu.CompilerParams(needs_layout_passes=False, use_tc_tiling_on_sc=False),
    out_type=jax.ShapeDtypeStruct((E, F_IN // 2), jnp.int32),
    scratch_types=[
        pltpu.VMEM((CH,), jnp.int32),
        pltpu.VMEM((CH, F_IN // 2), jnp.int32),
        pltpu.SemaphoreType.DMA,
    ],
)
def _sc_gather_x(x_hbm, src_hbm, out_hbm, idx_v, rows_v, sem):
    base = _wid() * EPW

    def body(i, _):
        off = base + i * CH
        pltpu.sync_copy(src_hbm.at[pl.ds(off, CH)], idx_v)
        pltpu.async_copy(x_hbm.at[idx_v], rows_v, sem).wait()
        pltpu.sync_copy(rows_v, out_hbm.at[pl.ds(off, CH)])
        return ()

    lax.fori_loop(0, NCH, body, ())


def _msg_body(xs_ref, ea_ref, wkn1_ref, bkn1_ref, wcat_ref, msg_ref):
    hT = jnp.broadcast_to(bkn1_ref[...], (KN, EB))
    for d in range(D_EDGE):
        hT = hT + wkn1_ref[:, d:d + 1] * ea_ref[d:d + 1, :]
    hT = jnp.maximum(hT, 0.0)
    xi = xs_ref[...]
    xe = lax.bitcast_convert_type(xi << 16, jnp.float32).astype(jnp.bfloat16)
    xo = lax.bitcast_convert_type(xi & jnp.int32(-65536),
                                  jnp.float32).astype(jnp.bfloat16)
    dims = (((0,), (1,)), ((), ()))
    PT = (jax.lax.dot_general(wcat_ref[0:F_IN // 2], xe, dims,
                              preferred_element_type=jnp.float32)
          + jax.lax.dot_general(wcat_ref[F_IN // 2:F_IN], xo, dims,
                                preferred_element_type=jnp.float32))
    acc = PT[0:F1, :]
    for k in range(KN):
        acc = acc + hT[k:k + 1, :] * PT[F1 * (k + 1):F1 * (k + 2), :]
    msg_ref[...] = acc.T


def _tc_msg(xs, eaT, wkn1T, bkn1c, wcat):
    return pl.pallas_call(
        _msg_body,
        grid=(GB,),
        in_specs=[
            pl.BlockSpec((EB, F_IN // 2), lambda i: (i, 0)),
            pl.BlockSpec((D_EDGE, EB), lambda i: (0, i)),
            pl.BlockSpec((KN, D_EDGE), lambda i: (0, 0)),
            pl.BlockSpec((KN, 1), lambda i: (0, 0)),
            pl.BlockSpec((F_IN, (KN + 1) * F1), lambda i: (0, 0)),
        ],
        out_specs=pl.BlockSpec((EB, F1), lambda i: (i, 0)),
        out_shape=jax.ShapeDtypeStruct((E, F1), jnp.float32),
    )(xs, eaT, wkn1T, bkn1c, wcat)


def _make_sc_scatter(width):
    @functools.partial(
        pl.kernel, mesh=_mesh,
        compiler_params=pltpu.CompilerParams(needs_layout_passes=False, use_tc_tiling_on_sc=False),
        out_type=jax.ShapeDtypeStruct((NC, N, width), jnp.float32),
        scratch_types=[
            pltpu.VMEM((CH,), jnp.int32),
            pltpu.VMEM((CH, width), jnp.float32),
            pltpu.VMEM_SHARED((N, width), jnp.float32),
        ],
    )
    def _sc_scatter(rows_hbm, dst_hbm, zeros_hbm, out_hbm, idx_v, rows_v, acc_sh):
        c = lax.axis_index("c")
        s = lax.axis_index("s")

        @pl.when(s == 0)
        def _():
            pltpu.sync_copy(zeros_hbm, acc_sh)

        plsc.subcore_barrier()
        base = (s * NC + c) * EPW

        def body(i, _):
            off = base + i * CH
            pltpu.sync_copy(dst_hbm.at[pl.ds(off, CH)], idx_v)
            pltpu.sync_copy(rows_hbm.at[pl.ds(off, CH)], rows_v)
            pltpu.sync_copy(rows_v, acc_sh.at[idx_v], add=True)
            return ()

        lax.fori_loop(0, NCH, body, ())
        plsc.subcore_barrier()

        @pl.when(s < 10)
        def _():
            pltpu.sync_copy(acc_sh.at[pl.ds(s * 1000, 1000)],
                            out_hbm.at[c, pl.ds(s * 1000, 1000)])

    return _sc_scatter


_sc_scatter64 = _make_sc_scatter(F1)


def _mid_body(agg_ref, x_ref, wroot_ref, becc_ref, wgat_ref, asv_ref, anv_ref,
              xp_ref, as_ref, an_ref, m_ref):
    agg = agg_ref[0] + agg_ref[1]
    x1 = jnp.maximum(
        agg + jnp.dot(x_ref[...], wroot_ref[...],
                      preferred_element_type=jnp.float32) + becc_ref[...], 0.0)
    xp = jnp.dot(x1, wgat_ref[...], preferred_element_type=jnp.float32)
    xp_ref[...] = xp
    a_s = jnp.dot(xp, asv_ref[...], preferred_element_type=jnp.float32)
    a_n = jnp.dot(xp, anv_ref[...], preferred_element_type=jnp.float32)
    as_ref[...] = a_s
    an_ref[...] = a_n
    t = jnp.max(a_s) + jnp.max(a_n)
    m_ref[...] = jnp.full((1, 1), jnp.where(t < 0.0, 0.2 * t, t))


def _tc_mid(agg2, x, W_root, b_ecc, W_gat, a_self, a_neigh):
    return pl.pallas_call(
        _mid_body,
        out_shape=(
            jax.ShapeDtypeStruct((N, F2), jnp.float32),
            jax.ShapeDtypeStruct((N, 1), jnp.float32),
            jax.ShapeDtypeStruct((N, 1), jnp.float32),
            jax.ShapeDtypeStruct((1, 1), jnp.float32),
        ),
    )(agg2, x, W_root, b_ecc.reshape(1, F1), W_gat,
      a_self.reshape(F2, 1), a_neigh.reshape(F2, 1))


@functools.partial(
    pl.kernel, mesh=_mesh,
    compiler_params=pltpu.CompilerParams(needs_layout_passes=False, use_tc_tiling_on_sc=False),
    out_type=(
        jax.ShapeDtypeStruct((E,), jnp.float32),
        jax.ShapeDtypeStruct((NC, N, W80), jnp.float32),
    ),
    scratch_types=[
        pltpu.VMEM((N,), jnp.float32),
        pltpu.VMEM((N,), jnp.float32),
        pltpu.VMEM((16,), jnp.float32),
        pltpu.VMEM((CH,), jnp.int32),
        pltpu.VMEM((CH,), jnp.int32),
        pltpu.VMEM((CH,), jnp.float32),
        pltpu.VMEM((CH, F2), jnp.float32),
        pltpu.VMEM((CH, W80), jnp.float32),
        pltpu.VMEM_SHARED((N, W80), jnp.float32),
        pltpu.SemaphoreType.DMA,
    ],
)
def _sc_edge(as_hbm, an_hbm, m_hbm, dst_hbm, src_hbm, xp_hbm, zeros_hbm,
             ex_hbm, acc_hbm,
             as_t, an_t, m_v, di_v, si_v, ex_v, rows_v, r80_v, acc_sh, sem):
    c = lax.axis_index("c")
    s = lax.axis_index("s")

    @pl.when(s == 0)
    def _():
        pltpu.sync_copy(zeros_hbm, acc_sh)

    pltpu.sync_copy(as_hbm, as_t)
    pltpu.sync_copy(an_hbm, an_t)
    pltpu.sync_copy(m_hbm, m_v)
    plsc.subcore_barrier()
    base = (s * NC + c) * EPW
    onehot = jnp.where(lax.iota(jnp.int32, 16) == 0, 1.0, 0.0)

    def body(i, _):
        off = base + i * CH
        pltpu.sync_copy(dst_hbm.at[pl.ds(off, CH)], di_v)
        pltpu.sync_copy(src_hbm.at[pl.ds(off, CH)], si_v)
        pltpu.async_copy(xp_hbm.at[si_v], rows_v, sem).wait()
        mvec = m_v[...]
        for j in range(CH // 16):
            dv = di_v[pl.ds(j * 16, 16)]
            sv = si_v[pl.ds(j * 16, 16)]
            l = plsc.load_gather(as_t, [dv]) + plsc.load_gather(an_t, [sv])
            l = jnp.where(l < 0.0, 0.2 * l, l)
            ex_v[pl.ds(j * 16, 16)] = jnp.exp(l - mvec)

        def scale(r, _):
            exb = plsc.load_gather(ex_v, [jnp.full((16,), r, jnp.int32)])
            for q in range(F2 // 16):
                r80_v[r, pl.ds(q * 16, 16)] = rows_v[r, pl.ds(q * 16, 16)] * exb
            r80_v[r, pl.ds(F2, 16)] = exb * onehot
            return ()

        lax.fori_loop(0, CH, scale, ())
        pltpu.sync_copy(r80_v, acc_sh.at[di_v], add=True)
        pltpu.sync_copy(ex_v, ex_hbm.at[pl.ds(off, CH)])
        return ()

    lax.fori_loop(0, NCH, body, ())
    plsc.subcore_barrier()

    @pl.when(s < 10)
    def _():
        pltpu.sync_copy(acc_sh.at[pl.ds(s * 1000, 1000)],
                        acc_hbm.at[c, pl.ds(s * 1000, 1000)])


def _fin_body(x2acc_ref, xp_ref, as_ref, an_ref, m_ref, bgat_ref,
              wfc_ref, bfc_ref, wout_ref, bout_ref,
              out_ref, attns_ref, inv_ref):
    acc = x2acc_ref[0] + x2acc_ref[1]
    t = as_ref[...] + an_ref[...]
    lse = jnp.where(t < 0.0, 0.2 * t, t)
    ex_self = jnp.exp(lse - m_ref[0, 0])
    denom = acc[:, F2:F2 + 1] + ex_self
    inv = 1.0 / denom
    inv_ref[...] = inv
    attns_ref[...] = ex_self * inv
    x2 = jnp.maximum((acc[:, 0:F2] + ex_self * xp_ref[...]) * inv
                     + bgat_ref[...], 0.0)
    p = jnp.mean(x2, axis=0, keepdims=True)
    f = jnp.maximum(jnp.dot(p, wfc_ref[...], preferred_element_type=jnp.float32)
                    + bfc_ref[...], 0.0)
    o = jnp.dot(f, wout_ref[...], preferred_element_type=jnp.float32) + bout_ref[...]
    out_ref[...] = jax.nn.sigmoid(o)


def _tc_final(x2acc, xp, a_s, a_n, m, b_gat, W_fc, b_fc, W_out, b_out):
    return pl.pallas_call(
        _fin_body,
        out_shape=(
            jax.ShapeDtypeStruct((1, 1), jnp.float32),
            jax.ShapeDtypeStruct((N, 1), jnp.float32),
            jax.ShapeDtypeStruct((N, 1), jnp.float32),
        ),
    )(x2acc, xp, a_s, a_n, m, b_gat.reshape(1, F2),
      W_fc, b_fc.reshape(1, 32), W_out, b_out.reshape(1, 1))


@functools.partial(
    pl.kernel, mesh=_mesh,
    compiler_params=pltpu.CompilerParams(needs_layout_passes=False, use_tc_tiling_on_sc=False),
    out_type=jax.ShapeDtypeStruct((E,), jnp.float32),
    scratch_types=[
        pltpu.VMEM((N,), jnp.float32),
        pltpu.VMEM((CH,), jnp.int32),
        pltpu.VMEM((CH,), jnp.float32),
        pltpu.VMEM((CH,), jnp.float32),
    ],
)
def _sc_attn(inv_hbm, dst_hbm, ex_hbm, attn_hbm, inv_t, di_v, ex_v, at_v):
    pltpu.sync_copy(inv_hbm, inv_t)
    base = _wid() * EPW

    def body(i, _):
        off = base + i * CH
        pltpu.sync_copy(dst_hbm.at[pl.ds(off, CH)], di_v)
        pltpu.sync_copy(ex_hbm.at[pl.ds(off, CH)], ex_v)
        for j in range(CH // 16):
            dv = di_v[pl.ds(j * 16, 16)]
            at_v[pl.ds(j * 16, 16)] = (ex_v[pl.ds(j * 16, 16)]
                                       * plsc.load_gather(inv_t, [dv]))
        pltpu.sync_copy(at_v, attn_hbm.at[pl.ds(off, CH)])
        return ()

    lax.fori_loop(0, NCH, body, ())


def kernel(x, edge_index, edge_attr, W_kn1, b_kn1, W_kn2, b_kn2, W_root, b_ecc,
           W_gat, a_self, a_neigh, b_gat, W_fc, b_fc, W_out, b_out):
    dst = edge_index[0]
    src = edge_index[1]

    Wk = W_kn2.reshape(KN, F_IN, F1)
    wcat = jnp.concatenate([b_kn2.reshape(1, F_IN, F1), Wk], axis=0)
    wcat = wcat.transpose(1, 0, 2).reshape(F_IN, (KN + 1) * F1)
    wcat = jnp.concatenate([wcat[0::2], wcat[1::2]], axis=0)
    wcat = wcat.astype(jnp.bfloat16)

    zeros64 = jnp.zeros((N, F1), jnp.float32)
    zeros80 = jnp.zeros((N, W80), jnp.float32)

    xpack = lax.bitcast_convert_type(
        x.astype(jnp.bfloat16).reshape(N, F_IN // 2, 2), jnp.int32)
    xs = _sc_gather_x(xpack, src)
    msg = _tc_msg(xs, edge_attr.T, W_kn1.T, b_kn1.reshape(KN, 1), wcat)
    agg2 = _sc_scatter64(msg, dst, zeros64)
    xp, a_s, a_n, m = _tc_mid(agg2, x, W_root, b_ecc, W_gat, a_self, a_neigh)

    ex, x2acc = _sc_edge(a_s.reshape(N), a_n.reshape(N),
                         jnp.broadcast_to(m.reshape(1), (16,)),
                         dst, src, xp, zeros80)
    out, attn_self, inv_denom = _tc_final(x2acc, xp, a_s, a_n, m, b_gat,
                                          W_fc, b_fc, W_out, b_out)
    attn_e = _sc_attn(inv_denom.reshape(N), dst, ex)

    attn = jnp.concatenate([attn_e, attn_self.reshape(N)])
    return (out.reshape(1), attn)


# speedup vs baseline: 1.3548x; 1.3548x over previous
import functools

import jax
import jax.numpy as jnp
from jax import lax
from jax.experimental import pallas as pl
from jax.experimental.pallas import tpu as pltpu
from jax.experimental.pallas import tpu_sc as plsc

N = 10000
E = 320000
F_IN = 128
D_EDGE = 4
KN = 32
F1 = 64
F2 = 64
W80 = 80

NC = 2
NS = 16
NW = NC * NS
EPW = E // NW
CH = 400
NCH = EPW // CH
RPT = N // NS

_mesh = plsc.VectorSubcoreMesh(core_axis_name="c", subcore_axis_name="s")
EB = 512
GB = E // EB


def _wid():
    return lax.axis_index("s") * NC + lax.axis_index("c")


H1 = 163840
H2 = E - H1
_PIPE = ((H1, 5120, 512, 10), (H2, 4880, 488, 10))


def _make_sc_gather(n_e, epw, ch, nch):
    @functools.partial(
        pl.kernel, mesh=_mesh,
        compiler_params=pltpu.CompilerParams(needs_layout_passes=False),
        out_type=jax.ShapeDtypeStruct((n_e, F_IN), jnp.float32),
        scratch_types=[
            pltpu.VMEM((ch,), jnp.int32),
            pltpu.VMEM((ch, F_IN), jnp.float32),
            pltpu.SemaphoreType.DMA,
        ],
    )
    def _sc_gather_x(x_hbm, src_hbm, out_hbm, idx_v, rows_v, sem):
        base = _wid() * epw

        def body(i, _):
            off = base + i * ch
            pltpu.sync_copy(src_hbm.at[pl.ds(off, ch)], idx_v)
            pltpu.async_copy(x_hbm.at[idx_v], rows_v, sem).wait()
            pltpu.sync_copy(rows_v, out_hbm.at[pl.ds(off, ch)])
            return ()

        lax.fori_loop(0, nch, body, ())

    return _sc_gather_x


_sc_gather_h = tuple(_make_sc_gather(*p) for p in _PIPE)


def _msg_body(xs_ref, ea_ref, wkn1_ref, bkn1_ref, wcat_ref, msg_ref):
    hT = jnp.broadcast_to(bkn1_ref[...], (KN, EB))
    for d in range(D_EDGE):
        hT = hT + wkn1_ref[:, d:d + 1] * ea_ref[d:d + 1, :]
    hT = jnp.maximum(hT, 0.0)
    xs = xs_ref[...].astype(jnp.bfloat16)
    PT = jax.lax.dot_general(wcat_ref[...], xs,
                             (((0,), (1,)), ((), ())),
                             preferred_element_type=jnp.float32)
    acc = PT[0:F1, :]
    for k in range(KN):
        acc = acc + hT[k:k + 1, :] * PT[F1 * (k + 1):F1 * (k + 2), :]
    msg_ref[...] = acc.T


def _tc_msg(xs, eaT, wkn1T, bkn1c, wcat):
    n_e = xs.shape[0]
    return pl.pallas_call(
        _msg_body,
        grid=(n_e // EB,),
        in_specs=[
            pl.BlockSpec((EB, F_IN), lambda i: (i, 0)),
            pl.BlockSpec((D_EDGE, EB), lambda i: (0, i)),
            pl.BlockSpec((KN, D_EDGE), lambda i: (0, 0)),
            pl.BlockSpec((KN, 1), lambda i: (0, 0)),
            pl.BlockSpec((F_IN, (KN + 1) * F1), lambda i: (0, 0)),
        ],
        out_specs=pl.BlockSpec((EB, F1), lambda i: (i, 0)),
        out_shape=jax.ShapeDtypeStruct((n_e, F1), jnp.float32),
    )(xs, eaT, wkn1T, bkn1c, wcat)


def _make_sc_scatter(width, epw, ch, nch):
    @functools.partial(
        pl.kernel, mesh=_mesh,
        compiler_params=pltpu.CompilerParams(needs_layout_passes=False, use_tc_tiling_on_sc=False),
        out_type=jax.ShapeDtypeStruct((NC, N, width), jnp.float32),
        scratch_types=[
            pltpu.VMEM((ch,), jnp.int32),
            pltpu.VMEM((ch, width), jnp.float32),
            pltpu.VMEM_SHARED((N, width), jnp.float32),
        ],
    )
    def _sc_scatter(rows_hbm, dst_hbm, zeros_hbm, out_hbm, idx_v, rows_v, acc_sh):
        c = lax.axis_index("c")
        s = lax.axis_index("s")

        @pl.when(s == 0)
        def _():
            pltpu.sync_copy(zeros_hbm, acc_sh)

        plsc.subcore_barrier()
        base = (s * NC + c) * epw

        def body(i, _):
            off = base + i * ch
            pltpu.sync_copy(dst_hbm.at[pl.ds(off, ch)], idx_v)
            pltpu.sync_copy(rows_hbm.at[pl.ds(off, ch)], rows_v)
            pltpu.sync_copy(rows_v, acc_sh.at[idx_v], add=True)
            return ()

        lax.fori_loop(0, nch, body, ())
        plsc.subcore_barrier()

        @pl.when(s < 10)
        def _():
            pltpu.sync_copy(acc_sh.at[pl.ds(s * 1000, 1000)],
                            out_hbm.at[c, pl.ds(s * 1000, 1000)])

    return _sc_scatter


_sc_scatter_h = tuple(_make_sc_scatter(F1, p[1], p[2], p[3]) for p in _PIPE)


def _mid_body(agga_ref, aggb_ref, x_ref, wroot_ref, becc_ref, wgat_ref,
              asv_ref, anv_ref, xp_ref, as_ref, an_ref, m_ref):
    agg = (agga_ref[0] + agga_ref[1]) + (aggb_ref[0] + aggb_ref[1])
    x1 = jnp.maximum(
        agg + jnp.dot(x_ref[...], wroot_ref[...],
                      preferred_element_type=jnp.float32) + becc_ref[...], 0.0)
    xp = jnp.dot(x1, wgat_ref[...], preferred_element_type=jnp.float32)
    xp_ref[...] = xp
    a_s = jnp.dot(xp, asv_ref[...], preferred_element_type=jnp.float32)
    a_n = jnp.dot(xp, anv_ref[...], preferred_element_type=jnp.float32)
    as_ref[...] = a_s
    an_ref[...] = a_n
    t = jnp.max(a_s) + jnp.max(a_n)
    m_ref[...] = jnp.full((1, 1), jnp.where(t < 0.0, 0.2 * t, t))


def _tc_mid(agg_a, agg_b, x, W_root, b_ecc, W_gat, a_self, a_neigh):
    return pl.pallas_call(
        _mid_body,
        out_shape=(
            jax.ShapeDtypeStruct((N, F2), jnp.float32),
            jax.ShapeDtypeStruct((N, 1), jnp.float32),
            jax.ShapeDtypeStruct((N, 1), jnp.float32),
            jax.ShapeDtypeStruct((1, 1), jnp.float32),
        ),
    )(agg_a, agg_b, x, W_root, b_ecc.reshape(1, F1), W_gat,
      a_self.reshape(F2, 1), a_neigh.reshape(F2, 1))


@functools.partial(
    pl.kernel, mesh=_mesh,
    compiler_params=pltpu.CompilerParams(needs_layout_passes=False, use_tc_tiling_on_sc=False),
    out_type=(
        jax.ShapeDtypeStruct((E,), jnp.float32),
        jax.ShapeDtypeStruct((NC, N, W80), jnp.float32),
    ),
    scratch_types=[
        pltpu.VMEM((N,), jnp.float32),
        pltpu.VMEM((N,), jnp.float32),
        pltpu.VMEM((16,), jnp.float32),
        pltpu.VMEM((CH,), jnp.int32),
        pltpu.VMEM((CH,), jnp.int32),
        pltpu.VMEM((CH,), jnp.float32),
        pltpu.VMEM((CH, F2), jnp.float32),
        pltpu.VMEM((CH, W80), jnp.float32),
        pltpu.VMEM_SHARED((N, W80), jnp.float32),
        pltpu.SemaphoreType.DMA,
    ],
)
def _sc_edge(as_hbm, an_hbm, m_hbm, dst_hbm, src_hbm, xp_hbm, zeros_hbm,
             ex_hbm, acc_hbm,
             as_t, an_t, m_v, di_v, si_v, ex_v, rows_v, r80_v, acc_sh, sem):
    c = lax.axis_index("c")
    s = lax.axis_index("s")

    @pl.when(s == 0)
    def _():
        pltpu.sync_copy(zeros_hbm, acc_sh)

    pltpu.sync_copy(as_hbm, as_t)
    pltpu.sync_copy(an_hbm, an_t)
    pltpu.sync_copy(m_hbm, m_v)
    plsc.subcore_barrier()
    base = (s * NC + c) * EPW
    onehot = jnp.where(lax.iota(jnp.int32, 16) == 0, 1.0, 0.0)

    def body(i, _):
        off = base + i * CH
        pltpu.sync_copy(dst_hbm.at[pl.ds(off, CH)], di_v)
        pltpu.sync_copy(src_hbm.at[pl.ds(off, CH)], si_v)
        pltpu.async_copy(xp_hbm.at[si_v], rows_v, sem).wait()
        mvec = m_v[...]
        for j in range(CH // 16):
            dv = di_v[pl.ds(j * 16, 16)]
            sv = si_v[pl.ds(j * 16, 16)]
            l = plsc.load_gather(as_t, [dv]) + plsc.load_gather(an_t, [sv])
            l = jnp.where(l < 0.0, 0.2 * l, l)
            ex_v[pl.ds(j * 16, 16)] = jnp.exp(l - mvec)

        def scale(r, _):
            exb = plsc.load_gather(ex_v, [jnp.full((16,), r, jnp.int32)])
            for q in range(F2 // 16):
                r80_v[r, pl.ds(q * 16, 16)] = rows_v[r, pl.ds(q * 16, 16)] * exb
            r80_v[r, pl.ds(F2, 16)] = exb * onehot
            return ()

        lax.fori_loop(0, CH, scale, ())
        pltpu.sync_copy(r80_v, acc_sh.at[di_v], add=True)
        pltpu.sync_copy(ex_v, ex_hbm.at[pl.ds(off, CH)])
        return ()

    lax.fori_loop(0, NCH, body, ())
    plsc.subcore_barrier()

    @pl.when(s < 10)
    def _():
        pltpu.sync_copy(acc_sh.at[pl.ds(s * 1000, 1000)],
                        acc_hbm.at[c, pl.ds(s * 1000, 1000)])


def _fin_body(x2acc_ref, xp_ref, as_ref, an_ref, m_ref, bgat_ref,
              wfc_ref, bfc_ref, wout_ref, bout_ref,
              out_ref, attns_ref, inv_ref):
    acc = x2acc_ref[0] + x2acc_ref[1]
    t = as_ref[...] + an_ref[...]
    lse = jnp.where(t < 0.0, 0.2 * t, t)
    ex_self = jnp.exp(lse - m_ref[0, 0])
    denom = acc[:, F2:F2 + 1] + ex_self
    inv = 1.0 / denom
    inv_ref[...] = inv
    attns_ref[...] = ex_self * inv
    x2 = jnp.maximum((acc[:, 0:F2] + ex_self * xp_ref[...]) * inv
                     + bgat_ref[...], 0.0)
    p = jnp.mean(x2, axis=0, keepdims=True)
    f = jnp.maximum(jnp.dot(p, wfc_ref[...], preferred_element_type=jnp.float32)
                    + bfc_ref[...], 0.0)
    o = jnp.dot(f, wout_ref[...], preferred_element_type=jnp.float32) + bout_ref[...]
    out_ref[...] = jax.nn.sigmoid(o)


def _tc_final(x2acc, xp, a_s, a_n, m, b_gat, W_fc, b_fc, W_out, b_out):
    return pl.pallas_call(
        _fin_body,
        out_shape=(
            jax.ShapeDtypeStruct((1, 1), jnp.float32),
            jax.ShapeDtypeStruct((N, 1), jnp.float32),
            jax.ShapeDtypeStruct((N, 1), jnp.float32),
        ),
    )(x2acc, xp, a_s, a_n, m, b_gat.reshape(1, F2),
      W_fc, b_fc.reshape(1, 32), W_out, b_out.reshape(1, 1))


@functools.partial(
    pl.kernel, mesh=_mesh,
    compiler_params=pltpu.CompilerParams(needs_layout_passes=False, use_tc_tiling_on_sc=False),
    out_type=jax.ShapeDtypeStruct((E,), jnp.float32),
    scratch_types=[
        pltpu.VMEM((N,), jnp.float32),
        pltpu.VMEM((CH,), jnp.int32),
        pltpu.VMEM((CH,), jnp.float32),
        pltpu.VMEM((CH,), jnp.float32),
    ],
)
def _sc_attn(inv_hbm, dst_hbm, ex_hbm, attn_hbm, inv_t, di_v, ex_v, at_v):
    pltpu.sync_copy(inv_hbm, inv_t)
    base = _wid() * EPW

    def body(i, _):
        off = base + i * CH
        pltpu.sync_copy(dst_hbm.at[pl.ds(off, CH)], di_v)
        pltpu.sync_copy(ex_hbm.at[pl.ds(off, CH)], ex_v)
        for j in range(CH // 16):
            dv = di_v[pl.ds(j * 16, 16)]
            at_v[pl.ds(j * 16, 16)] = (ex_v[pl.ds(j * 16, 16)]
                                       * plsc.load_gather(inv_t, [dv]))
        pltpu.sync_copy(at_v, attn_hbm.at[pl.ds(off, CH)])
        return ()

    lax.fori_loop(0, NCH, body, ())


def kernel(x, edge_index, edge_attr, W_kn1, b_kn1, W_kn2, b_kn2, W_root, b_ecc,
           W_gat, a_self, a_neigh, b_gat, W_fc, b_fc, W_out, b_out):
    dst = edge_index[0]
    src = edge_index[1]

    Wk = W_kn2.reshape(KN, F_IN, F1)
    wcat = jnp.concatenate([b_kn2.reshape(1, F_IN, F1), Wk], axis=0)
    wcat = wcat.transpose(1, 0, 2).reshape(F_IN, (KN + 1) * F1)
    wcat = wcat.astype(jnp.bfloat16)

    zeros64 = jnp.zeros((N, F1), jnp.float32)
    zeros80 = jnp.zeros((N, W80), jnp.float32)

    eaT = edge_attr.T
    wkn1T = W_kn1.T
    bkn1c = b_kn1.reshape(KN, 1)
    xs1 = _sc_gather_h[0](x, src[:H1])
    xs2 = _sc_gather_h[1](x, src[H1:])
    msg1 = _tc_msg(xs1, eaT[:, :H1], wkn1T, bkn1c, wcat)
    agg_a = _sc_scatter_h[0](msg1, dst[:H1], zeros64)
    msg2 = _tc_msg(xs2, eaT[:, H1:], wkn1T, bkn1c, wcat)
    agg_b = _sc_scatter_h[1](msg2, dst[H1:], zeros64)
    xp, a_s, a_n, m = _tc_mid(agg_a, agg_b, x, W_root, b_ecc, W_gat,
                              a_self, a_neigh)

    ex, x2acc = _sc_edge(a_s.reshape(N), a_n.reshape(N),
                         jnp.broadcast_to(m.reshape(1), (16,)),
                         dst, src, xp, zeros80)
    out, attn_self, inv_denom = _tc_final(x2acc, xp, a_s, a_n, m, b_gat,
                                          W_fc, b_fc, W_out, b_out)
    attn_e = _sc_attn(inv_denom.reshape(N), dst, ex)

    attn = jnp.concatenate([attn_e, attn_self.reshape(N)])
    return (out.reshape(1), attn)


# speedup vs baseline: 1.5787x; 1.1653x over previous
import functools

import jax
import jax.numpy as jnp
from jax import lax
from jax.experimental import pallas as pl
from jax.experimental.pallas import tpu as pltpu
from jax.experimental.pallas import tpu_sc as plsc

N = 10000
E = 320000
F_IN = 128
D_EDGE = 4
KN = 32
F1 = 64
F2 = 64
W80 = 80

NC = 2
NS = 16
NW = NC * NS
EPW = E // NW
CH = 400
NCH = EPW // CH
RPT = N // NS

_mesh = plsc.VectorSubcoreMesh(core_axis_name="c", subcore_axis_name="s")
EB = 1280


def _wid():
    return lax.axis_index("s") * NC + lax.axis_index("c")


H1 = 163840
H2 = E - H1
_PIPE = ((H1, 5120, 512, 10), (H2, 4880, 488, 10))


def _make_sc_gather(n_e, epw, ch, nch):
    @functools.partial(
        pl.kernel, mesh=_mesh,
        compiler_params=pltpu.CompilerParams(needs_layout_passes=False),
        out_type=jax.ShapeDtypeStruct((n_e, F_IN), jnp.float32),
        scratch_types=[
            pltpu.VMEM((ch,), jnp.int32),
            pltpu.VMEM((ch, F_IN), jnp.float32),
            pltpu.SemaphoreType.DMA,
        ],
    )
    def _sc_gather_x(x_hbm, src_hbm, out_hbm, idx_v, rows_v, sem):
        base = _wid() * epw

        def body(i, _):
            off = base + i * ch
            pltpu.sync_copy(src_hbm.at[pl.ds(off, ch)], idx_v)
            pltpu.async_copy(x_hbm.at[idx_v], rows_v, sem).wait()
            pltpu.sync_copy(rows_v, out_hbm.at[pl.ds(off, ch)])
            return ()

        lax.fori_loop(0, nch, body, ())

    return _sc_gather_x


_sc_gather_h = tuple(_make_sc_gather(*p) for p in _PIPE)


def _msg_body(xs_ref, ea_ref, wkn1_ref, bkn1_ref, wcat_ref, msg_ref):
    hT = jnp.broadcast_to(bkn1_ref[...], (KN, EB))
    for d in range(D_EDGE):
        hT = hT + wkn1_ref[:, d:d + 1] * ea_ref[d:d + 1, :]
    hT = jnp.maximum(hT, 0.0)
    xs = xs_ref[...].astype(jnp.bfloat16)
    PT = jax.lax.dot_general(wcat_ref[...], xs,
                             (((0,), (1,)), ((), ())),
                             preferred_element_type=jnp.float32)
    acc = PT[0:F1, :]
    for k in range(KN):
        acc = acc + hT[k:k + 1, :] * PT[F1 * (k + 1):F1 * (k + 2), :]
    msg_ref[...] = acc.T


def _tc_msg(xs, eaT, wkn1T, bkn1c, wcat):
    n_e = xs.shape[0]
    return pl.pallas_call(
        _msg_body,
        grid=(n_e // EB,),
        in_specs=[
            pl.BlockSpec((EB, F_IN), lambda i: (i, 0)),
            pl.BlockSpec((D_EDGE, EB), lambda i: (0, i)),
            pl.BlockSpec((KN, D_EDGE), lambda i: (0, 0)),
            pl.BlockSpec((KN, 1), lambda i: (0, 0)),
            pl.BlockSpec((F_IN, (KN + 1) * F1), lambda i: (0, 0)),
        ],
        out_specs=pl.BlockSpec((EB, F1), lambda i: (i, 0)),
        out_shape=jax.ShapeDtypeStruct((n_e, F1), jnp.float32),
    )(xs, eaT, wkn1T, bkn1c, wcat)


def _make_sc_scatter(width, epw, ch, nch):
    @functools.partial(
        pl.kernel, mesh=_mesh,
        compiler_params=pltpu.CompilerParams(needs_layout_passes=False, use_tc_tiling_on_sc=False),
        out_type=jax.ShapeDtypeStruct((NC, N, width), jnp.float32),
        scratch_types=[
            pltpu.VMEM((ch,), jnp.int32),
            pltpu.VMEM((ch, width), jnp.float32),
            pltpu.VMEM_SHARED((N, width), jnp.float32),
        ],
    )
    def _sc_scatter(rows_hbm, dst_hbm, zeros_hbm, out_hbm, idx_v, rows_v, acc_sh):
        c = lax.axis_index("c")
        s = lax.axis_index("s")

        @pl.when(s == 0)
        def _():
            pltpu.sync_copy(zeros_hbm, acc_sh)

        plsc.subcore_barrier()
        base = (s * NC + c) * epw

        def body(i, _):
            off = base + i * ch
            pltpu.sync_copy(dst_hbm.at[pl.ds(off, ch)], idx_v)
            pltpu.sync_copy(rows_hbm.at[pl.ds(off, ch)], rows_v)
            pltpu.sync_copy(rows_v, acc_sh.at[idx_v], add=True)
            return ()

        lax.fori_loop(0, nch, body, ())
        plsc.subcore_barrier()

        @pl.when(s < 10)
        def _():
            pltpu.sync_copy(acc_sh.at[pl.ds(s * 1000, 1000)],
                            out_hbm.at[c, pl.ds(s * 1000, 1000)])

    return _sc_scatter


_sc_scatter_h = tuple(_make_sc_scatter(F1, p[1], p[2], p[3]) for p in _PIPE)


def _mid_body(agga_ref, aggb_ref, x_ref, wroot_ref, becc_ref, wgat_ref,
              asv_ref, anv_ref, xp_ref, as_ref, an_ref, m_ref):
    agg = (agga_ref[0] + agga_ref[1]) + (aggb_ref[0] + aggb_ref[1])
    x1 = jnp.maximum(
        agg + jnp.dot(x_ref[...], wroot_ref[...],
                      preferred_element_type=jnp.float32) + becc_ref[...], 0.0)
    xp = jnp.dot(x1, wgat_ref[...], preferred_element_type=jnp.float32)
    xp_ref[...] = xp
    a_s = jnp.dot(xp, asv_ref[...], preferred_element_type=jnp.float32)
    a_n = jnp.dot(xp, anv_ref[...], preferred_element_type=jnp.float32)
    as_ref[...] = a_s
    an_ref[...] = a_n
    t = jnp.max(a_s) + jnp.max(a_n)
    m_ref[...] = jnp.full((1, 1), jnp.where(t < 0.0, 0.2 * t, t))


def _tc_mid(agg_a, agg_b, x, W_root, b_ecc, W_gat, a_self, a_neigh):
    return pl.pallas_call(
        _mid_body,
        out_shape=(
            jax.ShapeDtypeStruct((N, F2), jnp.float32),
            jax.ShapeDtypeStruct((N, 1), jnp.float32),
            jax.ShapeDtypeStruct((N, 1), jnp.float32),
            jax.ShapeDtypeStruct((1, 1), jnp.float32),
        ),
    )(agg_a, agg_b, x, W_root, b_ecc.reshape(1, F1), W_gat,
      a_self.reshape(F2, 1), a_neigh.reshape(F2, 1))


@functools.partial(
    pl.kernel, mesh=_mesh,
    compiler_params=pltpu.CompilerParams(needs_layout_passes=False, use_tc_tiling_on_sc=False),
    out_type=(
        jax.ShapeDtypeStruct((E,), jnp.float32),
        jax.ShapeDtypeStruct((NC, N, W80), jnp.float32),
    ),
    scratch_types=[
        pltpu.VMEM((N,), jnp.float32),
        pltpu.VMEM((N,), jnp.float32),
        pltpu.VMEM((16,), jnp.float32),
        pltpu.VMEM((CH,), jnp.int32),
        pltpu.VMEM((CH,), jnp.int32),
        pltpu.VMEM((CH,), jnp.float32),
        pltpu.VMEM((CH, F2), jnp.float32),
        pltpu.VMEM((CH, W80), jnp.float32),
        pltpu.VMEM_SHARED((N, W80), jnp.float32),
        pltpu.SemaphoreType.DMA,
    ],
)
def _sc_edge(as_hbm, an_hbm, m_hbm, dst_hbm, src_hbm, xp_hbm, zeros_hbm,
             ex_hbm, acc_hbm,
             as_t, an_t, m_v, di_v, si_v, ex_v, rows_v, r80_v, acc_sh, sem):
    c = lax.axis_index("c")
    s = lax.axis_index("s")

    @pl.when(s == 0)
    def _():
        pltpu.sync_copy(zeros_hbm, acc_sh)

    pltpu.sync_copy(as_hbm, as_t)
    pltpu.sync_copy(an_hbm, an_t)
    pltpu.sync_copy(m_hbm, m_v)
    plsc.subcore_barrier()
    base = (s * NC + c) * EPW
    onehot = jnp.where(lax.iota(jnp.int32, 16) == 0, 1.0, 0.0)

    def body(i, _):
        off = base + i * CH
        pltpu.sync_copy(dst_hbm.at[pl.ds(off, CH)], di_v)
        pltpu.sync_copy(src_hbm.at[pl.ds(off, CH)], si_v)
        pltpu.async_copy(xp_hbm.at[si_v], rows_v, sem).wait()
        mvec = m_v[...]
        for j in range(CH // 16):
            dv = di_v[pl.ds(j * 16, 16)]
            sv = si_v[pl.ds(j * 16, 16)]
            l = plsc.load_gather(as_t, [dv]) + plsc.load_gather(an_t, [sv])
            l = jnp.where(l < 0.0, 0.2 * l, l)
            ex_v[pl.ds(j * 16, 16)] = jnp.exp(l - mvec)

        def scale(r, _):
            exb = plsc.load_gather(ex_v, [jnp.full((16,), r, jnp.int32)])
            for q in range(F2 // 16):
                r80_v[r, pl.ds(q * 16, 16)] = rows_v[r, pl.ds(q * 16, 16)] * exb
            r80_v[r, pl.ds(F2, 16)] = exb * onehot
            return ()

        lax.fori_loop(0, CH, scale, ())
        pltpu.sync_copy(r80_v, acc_sh.at[di_v], add=True)
        pltpu.sync_copy(ex_v, ex_hbm.at[pl.ds(off, CH)])
        return ()

    lax.fori_loop(0, NCH, body, ())
    plsc.subcore_barrier()

    @pl.when(s < 10)
    def _():
        pltpu.sync_copy(acc_sh.at[pl.ds(s * 1000, 1000)],
                        acc_hbm.at[c, pl.ds(s * 1000, 1000)])


def _fin_body(x2acc_ref, xp_ref, as_ref, an_ref, m_ref, bgat_ref,
              wfc_ref, bfc_ref, wout_ref, bout_ref,
              out_ref, attns_ref, inv_ref):
    acc = x2acc_ref[0] + x2acc_ref[1]
    t = as_ref[...] + an_ref[...]
    lse = jnp.where(t < 0.0, 0.2 * t, t)
    ex_self = jnp.exp(lse - m_ref[0, 0])
    denom = acc[:, F2:F2 + 1] + ex_self
    inv = 1.0 / denom
    inv_ref[...] = inv
    attns_ref[...] = ex_self * inv
    x2 = jnp.maximum((acc[:, 0:F2] + ex_self * xp_ref[...]) * inv
                     + bgat_ref[...], 0.0)
    p = jnp.mean(x2, axis=0, keepdims=True)
    f = jnp.maximum(jnp.dot(p, wfc_ref[...], preferred_element_type=jnp.float32)
                    + bfc_ref[...], 0.0)
    o = jnp.dot(f, wout_ref[...], preferred_element_type=jnp.float32) + bout_ref[...]
    out_ref[...] = jax.nn.sigmoid(o)


def _tc_final(x2acc, xp, a_s, a_n, m, b_gat, W_fc, b_fc, W_out, b_out):
    return pl.pallas_call(
        _fin_body,
        out_shape=(
            jax.ShapeDtypeStruct((1, 1), jnp.float32),
            jax.ShapeDtypeStruct((N, 1), jnp.float32),
            jax.ShapeDtypeStruct((N, 1), jnp.float32),
        ),
    )(x2acc, xp, a_s, a_n, m, b_gat.reshape(1, F2),
      W_fc, b_fc.reshape(1, 32), W_out, b_out.reshape(1, 1))


@functools.partial(
    pl.kernel, mesh=_mesh,
    compiler_params=pltpu.CompilerParams(needs_layout_passes=False, use_tc_tiling_on_sc=False),
    out_type=jax.ShapeDtypeStruct((E,), jnp.float32),
    scratch_types=[
        pltpu.VMEM((N,), jnp.float32),
        pltpu.VMEM((CH,), jnp.int32),
        pltpu.VMEM((CH,), jnp.float32),
        pltpu.VMEM((CH,), jnp.float32),
    ],
)
def _sc_attn(inv_hbm, dst_hbm, ex_hbm, attn_hbm, inv_t, di_v, ex_v, at_v):
    pltpu.sync_copy(inv_hbm, inv_t)
    base = _wid() * EPW

    def body(i, _):
        off = base + i * CH
        pltpu.sync_copy(dst_hbm.at[pl.ds(off, CH)], di_v)
        pltpu.sync_copy(ex_hbm.at[pl.ds(off, CH)], ex_v)
        for j in range(CH // 16):
            dv = di_v[pl.ds(j * 16, 16)]
            at_v[pl.ds(j * 16, 16)] = (ex_v[pl.ds(j * 16, 16)]
                                       * plsc.load_gather(inv_t, [dv]))
        pltpu.sync_copy(at_v, attn_hbm.at[pl.ds(off, CH)])
        return ()

    lax.fori_loop(0, NCH, body, ())


def kernel(x, edge_index, edge_attr, W_kn1, b_kn1, W_kn2, b_kn2, W_root, b_ecc,
           W_gat, a_self, a_neigh, b_gat, W_fc, b_fc, W_out, b_out):
    dst = edge_index[0]
    src = edge_index[1]

    Wk = W_kn2.reshape(KN, F_IN, F1)
    wcat = jnp.concatenate([b_kn2.reshape(1, F_IN, F1), Wk], axis=0)
    wcat = wcat.transpose(1, 0, 2).reshape(F_IN, (KN + 1) * F1)
    wcat = wcat.astype(jnp.bfloat16)

    zeros64 = jnp.zeros((N, F1), jnp.float32)
    zeros80 = jnp.zeros((N, W80), jnp.float32)

    eaT = edge_attr.T
    wkn1T = W_kn1.T
    bkn1c = b_kn1.reshape(KN, 1)
    xs1 = _sc_gather_h[0](x, src[:H1])
    xs2 = _sc_gather_h[1](x, src[H1:])
    msg1 = _tc_msg(xs1, eaT[:, :H1], wkn1T, bkn1c, wcat)
    agg_a = _sc_scatter_h[0](msg1, dst[:H1], zeros64)
    msg2 = _tc_msg(xs2, eaT[:, H1:], wkn1T, bkn1c, wcat)
    agg_b = _sc_scatter_h[1](msg2, dst[H1:], zeros64)
    xp, a_s, a_n, m = _tc_mid(agg_a, agg_b, x, W_root, b_ecc, W_gat,
                              a_self, a_neigh)

    ex, x2acc = _sc_edge(a_s.reshape(N), a_n.reshape(N),
                         jnp.broadcast_to(m.reshape(1), (16,)),
                         dst, src, xp, zeros80)
    out, attn_self, inv_denom = _tc_final(x2acc, xp, a_s, a_n, m, b_gat,
                                          W_fc, b_fc, W_out, b_out)
    attn_e = _sc_attn(inv_denom.reshape(N), dst, ex)

    attn = jnp.concatenate([attn_e, attn_self.reshape(N)])
    return (out.reshape(1), attn)


# speedup vs baseline: 1.6636x; 1.0537x over previous
import functools

import jax
import jax.numpy as jnp
from jax import lax
from jax.experimental import pallas as pl
from jax.experimental.pallas import tpu as pltpu
from jax.experimental.pallas import tpu_sc as plsc

N = 10000
E = 320000
F_IN = 128
D_EDGE = 4
KN = 32
F1 = 64
F2 = 64
W80 = 80

NC = 2
NS = 16
NW = NC * NS
EPW = E // NW
CH = 400
NCH = EPW // CH
RPT = N // NS

_mesh = plsc.VectorSubcoreMesh(core_axis_name="c", subcore_axis_name="s")
EB = 2560


def _wid():
    return lax.axis_index("s") * NC + lax.axis_index("c")


H1 = 163840
H2 = E - H1
_PIPE = ((H1, 5120, 512, 10), (H2, 4880, 488, 10))


def _make_sc_gather(n_e, epw, ch, nch):
    @functools.partial(
        pl.kernel, mesh=_mesh,
        compiler_params=pltpu.CompilerParams(needs_layout_passes=False),
        out_type=jax.ShapeDtypeStruct((n_e, F_IN), jnp.float32),
        scratch_types=[
            pltpu.VMEM((ch,), jnp.int32),
            pltpu.VMEM((ch, F_IN), jnp.float32),
            pltpu.SemaphoreType.DMA,
        ],
    )
    def _sc_gather_x(x_hbm, src_hbm, out_hbm, idx_v, rows_v, sem):
        base = _wid() * epw

        def body(i, _):
            off = base + i * ch
            pltpu.sync_copy(src_hbm.at[pl.ds(off, ch)], idx_v)
            pltpu.async_copy(x_hbm.at[idx_v], rows_v, sem).wait()
            pltpu.sync_copy(rows_v, out_hbm.at[pl.ds(off, ch)])
            return ()

        lax.fori_loop(0, nch, body, ())

    return _sc_gather_x


_sc_gather_h = tuple(_make_sc_gather(*p) for p in _PIPE)


def _msg_body(xs_ref, ea_ref, wkn1_ref, bkn1_ref, wcat_ref, msg_ref):
    hT = jnp.broadcast_to(bkn1_ref[...], (KN, EB))
    for d in range(D_EDGE):
        hT = hT + wkn1_ref[:, d:d + 1] * ea_ref[d:d + 1, :]
    hT = jnp.maximum(hT, 0.0)
    xs = xs_ref[...].astype(jnp.bfloat16)
    PT = jax.lax.dot_general(wcat_ref[...], xs,
                             (((0,), (1,)), ((), ())),
                             preferred_element_type=jnp.float32)
    acc = PT[0:F1, :]
    for k in range(KN):
        acc = acc + hT[k:k + 1, :] * PT[F1 * (k + 1):F1 * (k + 2), :]
    msg_ref[...] = acc.T


def _tc_msg(xs, eaT, wkn1T, bkn1c, wcat):
    n_e = xs.shape[0]
    return pl.pallas_call(
        _msg_body,
        grid=(n_e // EB,),
        in_specs=[
            pl.BlockSpec((EB, F_IN), lambda i: (i, 0)),
            pl.BlockSpec((D_EDGE, EB), lambda i: (0, i)),
            pl.BlockSpec((KN, D_EDGE), lambda i: (0, 0)),
            pl.BlockSpec((KN, 1), lambda i: (0, 0)),
            pl.BlockSpec((F_IN, (KN + 1) * F1), lambda i: (0, 0)),
        ],
        out_specs=pl.BlockSpec((EB, F1), lambda i: (i, 0)),
        out_shape=jax.ShapeDtypeStruct((n_e, F1), jnp.float32),
    )(xs, eaT, wkn1T, bkn1c, wcat)


def _make_sc_scatter(width, epw, ch, nch):
    @functools.partial(
        pl.kernel, mesh=_mesh,
        compiler_params=pltpu.CompilerParams(needs_layout_passes=False, use_tc_tiling_on_sc=False),
        out_type=jax.ShapeDtypeStruct((NC, N, width), jnp.float32),
        scratch_types=[
            pltpu.VMEM((ch,), jnp.int32),
            pltpu.VMEM((ch, width), jnp.float32),
            pltpu.VMEM_SHARED((N, width), jnp.float32),
        ],
    )
    def _sc_scatter(rows_hbm, dst_hbm, zeros_hbm, out_hbm, idx_v, rows_v, acc_sh):
        c = lax.axis_index("c")
        s = lax.axis_index("s")

        @pl.when(s == 0)
        def _():
            pltpu.sync_copy(zeros_hbm, acc_sh)

        plsc.subcore_barrier()
        base = (s * NC + c) * epw

        def body(i, _):
            off = base + i * ch
            pltpu.sync_copy(dst_hbm.at[pl.ds(off, ch)], idx_v)
            pltpu.sync_copy(rows_hbm.at[pl.ds(off, ch)], rows_v)
            pltpu.sync_copy(rows_v, acc_sh.at[idx_v], add=True)
            return ()

        lax.fori_loop(0, nch, body, ())
        plsc.subcore_barrier()

        @pl.when(s < 10)
        def _():
            pltpu.sync_copy(acc_sh.at[pl.ds(s * 1000, 1000)],
                            out_hbm.at[c, pl.ds(s * 1000, 1000)])

    return _sc_scatter


_sc_scatter_h = tuple(_make_sc_scatter(F1, p[1], p[2], p[3]) for p in _PIPE)


def _mid_body(agga_ref, aggb_ref, x_ref, wroot_ref, becc_ref, wgat_ref,
              asv_ref, anv_ref, xp_ref, as_ref, an_ref, m_ref):
    agg = (agga_ref[0] + agga_ref[1]) + (aggb_ref[0] + aggb_ref[1])
    x1 = jnp.maximum(
        agg + jnp.dot(x_ref[...], wroot_ref[...],
                      preferred_element_type=jnp.float32) + becc_ref[...], 0.0)
    xp = jnp.dot(x1, wgat_ref[...], preferred_element_type=jnp.float32)
    xp_ref[...] = xp
    a_s = jnp.dot(xp, asv_ref[...], preferred_element_type=jnp.float32)
    a_n = jnp.dot(xp, anv_ref[...], preferred_element_type=jnp.float32)
    as_ref[...] = a_s
    an_ref[...] = a_n
    t = jnp.max(a_s) + jnp.max(a_n)
    m_ref[...] = jnp.full((1, 1), jnp.where(t < 0.0, 0.2 * t, t))


def _tc_mid(agg_a, agg_b, x, W_root, b_ecc, W_gat, a_self, a_neigh):
    return pl.pallas_call(
        _mid_body,
        out_shape=(
            jax.ShapeDtypeStruct((N, F2), jnp.float32),
            jax.ShapeDtypeStruct((N, 1), jnp.float32),
            jax.ShapeDtypeStruct((N, 1), jnp.float32),
            jax.ShapeDtypeStruct((1, 1), jnp.float32),
        ),
    )(agg_a, agg_b, x, W_root, b_ecc.reshape(1, F1), W_gat,
      a_self.reshape(F2, 1), a_neigh.reshape(F2, 1))


@functools.partial(
    pl.kernel, mesh=_mesh,
    compiler_params=pltpu.CompilerParams(needs_layout_passes=False, use_tc_tiling_on_sc=False),
    out_type=(
        jax.ShapeDtypeStruct((E,), jnp.float32),
        jax.ShapeDtypeStruct((NC, N, W80), jnp.float32),
    ),
    scratch_types=[
        pltpu.VMEM((N,), jnp.float32),
        pltpu.VMEM((N,), jnp.float32),
        pltpu.VMEM((16,), jnp.float32),
        pltpu.VMEM((CH,), jnp.int32),
        pltpu.VMEM((CH,), jnp.int32),
        pltpu.VMEM((CH,), jnp.float32),
        pltpu.VMEM((CH, F2), jnp.float32),
        pltpu.VMEM((CH, W80), jnp.float32),
        pltpu.VMEM_SHARED((N, W80), jnp.float32),
        pltpu.SemaphoreType.DMA,
    ],
)
def _sc_edge(as_hbm, an_hbm, m_hbm, dst_hbm, src_hbm, xp_hbm, zeros_hbm,
             ex_hbm, acc_hbm,
             as_t, an_t, m_v, di_v, si_v, ex_v, rows_v, r80_v, acc_sh, sem):
    c = lax.axis_index("c")
    s = lax.axis_index("s")

    @pl.when(s == 0)
    def _():
        pltpu.sync_copy(zeros_hbm, acc_sh)

    pltpu.sync_copy(as_hbm, as_t)
    pltpu.sync_copy(an_hbm, an_t)
    pltpu.sync_copy(m_hbm, m_v)
    plsc.subcore_barrier()
    base = (s * NC + c) * EPW
    onehot = jnp.where(lax.iota(jnp.int32, 16) == 0, 1.0, 0.0)

    def body(i, _):
        off = base + i * CH
        pltpu.sync_copy(dst_hbm.at[pl.ds(off, CH)], di_v)
        pltpu.sync_copy(src_hbm.at[pl.ds(off, CH)], si_v)
        pltpu.async_copy(xp_hbm.at[si_v], rows_v, sem).wait()
        mvec = m_v[...]
        for j in range(CH // 16):
            dv = di_v[pl.ds(j * 16, 16)]
            sv = si_v[pl.ds(j * 16, 16)]
            l = plsc.load_gather(as_t, [dv]) + plsc.load_gather(an_t, [sv])
            l = jnp.where(l < 0.0, 0.2 * l, l)
            ex_v[pl.ds(j * 16, 16)] = jnp.exp(l - mvec)

        def scale(r, _):
            exb = plsc.load_gather(ex_v, [jnp.full((16,), r, jnp.int32)])
            for q in range(F2 // 16):
                r80_v[r, pl.ds(q * 16, 16)] = rows_v[r, pl.ds(q * 16, 16)] * exb
            r80_v[r, pl.ds(F2, 16)] = exb * onehot
            return ()

        lax.fori_loop(0, CH, scale, ())
        pltpu.sync_copy(r80_v, acc_sh.at[di_v], add=True)
        pltpu.sync_copy(ex_v, ex_hbm.at[pl.ds(off, CH)])
        return ()

    lax.fori_loop(0, NCH, body, ())
    plsc.subcore_barrier()

    @pl.when(s < 10)
    def _():
        pltpu.sync_copy(acc_sh.at[pl.ds(s * 1000, 1000)],
                        acc_hbm.at[c, pl.ds(s * 1000, 1000)])


def _fin_body(x2acc_ref, xp_ref, as_ref, an_ref, m_ref, bgat_ref,
              wfc_ref, bfc_ref, wout_ref, bout_ref,
              out_ref, attns_ref, inv_ref):
    acc = x2acc_ref[0] + x2acc_ref[1]
    t = as_ref[...] + an_ref[...]
    lse = jnp.where(t < 0.0, 0.2 * t, t)
    ex_self = jnp.exp(lse - m_ref[0, 0])
    denom = acc[:, F2:F2 + 1] + ex_self
    inv = 1.0 / denom
    inv_ref[...] = inv
    attns_ref[...] = ex_self * inv
    x2 = jnp.maximum((acc[:, 0:F2] + ex_self * xp_ref[...]) * inv
                     + bgat_ref[...], 0.0)
    p = jnp.mean(x2, axis=0, keepdims=True)
    f = jnp.maximum(jnp.dot(p, wfc_ref[...], preferred_element_type=jnp.float32)
                    + bfc_ref[...], 0.0)
    o = jnp.dot(f, wout_ref[...], preferred_element_type=jnp.float32) + bout_ref[...]
    out_ref[...] = jax.nn.sigmoid(o)


def _tc_final(x2acc, xp, a_s, a_n, m, b_gat, W_fc, b_fc, W_out, b_out):
    return pl.pallas_call(
        _fin_body,
        out_shape=(
            jax.ShapeDtypeStruct((1, 1), jnp.float32),
            jax.ShapeDtypeStruct((N, 1), jnp.float32),
            jax.ShapeDtypeStruct((N, 1), jnp.float32),
        ),
    )(x2acc, xp, a_s, a_n, m, b_gat.reshape(1, F2),
      W_fc, b_fc.reshape(1, 32), W_out, b_out.reshape(1, 1))


@functools.partial(
    pl.kernel, mesh=_mesh,
    compiler_params=pltpu.CompilerParams(needs_layout_passes=False, use_tc_tiling_on_sc=False),
    out_type=jax.ShapeDtypeStruct((E,), jnp.float32),
    scratch_types=[
        pltpu.VMEM((N,), jnp.float32),
        pltpu.VMEM((CH,), jnp.int32),
        pltpu.VMEM((CH,), jnp.float32),
        pltpu.VMEM((CH,), jnp.float32),
    ],
)
def _sc_attn(inv_hbm, dst_hbm, ex_hbm, attn_hbm, inv_t, di_v, ex_v, at_v):
    pltpu.sync_copy(inv_hbm, inv_t)
    base = _wid() * EPW

    def body(i, _):
        off = base + i * CH
        pltpu.sync_copy(dst_hbm.at[pl.ds(off, CH)], di_v)
        pltpu.sync_copy(ex_hbm.at[pl.ds(off, CH)], ex_v)
        for j in range(CH // 16):
            dv = di_v[pl.ds(j * 16, 16)]
            at_v[pl.ds(j * 16, 16)] = (ex_v[pl.ds(j * 16, 16)]
                                       * plsc.load_gather(inv_t, [dv]))
        pltpu.sync_copy(at_v, attn_hbm.at[pl.ds(off, CH)])
        return ()

    lax.fori_loop(0, NCH, body, ())


def kernel(x, edge_index, edge_attr, W_kn1, b_kn1, W_kn2, b_kn2, W_root, b_ecc,
           W_gat, a_self, a_neigh, b_gat, W_fc, b_fc, W_out, b_out):
    dst = edge_index[0]
    src = edge_index[1]

    Wk = W_kn2.reshape(KN, F_IN, F1)
    wcat = jnp.concatenate([b_kn2.reshape(1, F_IN, F1), Wk], axis=0)
    wcat = wcat.transpose(1, 0, 2).reshape(F_IN, (KN + 1) * F1)
    wcat = wcat.astype(jnp.bfloat16)

    zeros64 = jnp.zeros((N, F1), jnp.float32)
    zeros80 = jnp.zeros((N, W80), jnp.float32)

    eaT = edge_attr.T
    wkn1T = W_kn1.T
    bkn1c = b_kn1.reshape(KN, 1)
    xs1 = _sc_gather_h[0](x, src[:H1])
    xs2 = _sc_gather_h[1](x, src[H1:])
    msg1 = _tc_msg(xs1, eaT[:, :H1], wkn1T, bkn1c, wcat)
    agg_a = _sc_scatter_h[0](msg1, dst[:H1], zeros64)
    msg2 = _tc_msg(xs2, eaT[:, H1:], wkn1T, bkn1c, wcat)
    agg_b = _sc_scatter_h[1](msg2, dst[H1:], zeros64)
    xp, a_s, a_n, m = _tc_mid(agg_a, agg_b, x, W_root, b_ecc, W_gat,
                              a_self, a_neigh)

    ex, x2acc = _sc_edge(a_s.reshape(N), a_n.reshape(N),
                         jnp.broadcast_to(m.reshape(1), (16,)),
                         dst, src, xp, zeros80)
    out, attn_self, inv_denom = _tc_final(x2acc, xp, a_s, a_n, m, b_gat,
                                          W_fc, b_fc, W_out, b_out)
    attn_e = _sc_attn(inv_denom.reshape(N), dst, ex)

    attn = jnp.concatenate([attn_e, attn_self.reshape(N)])
    return (out.reshape(1), attn)
